# baseline (speedup 1.0000x reference)
import functools

import jax
import jax.numpy as jnp
import numpy as np
from jax import lax
from jax.experimental import pallas as pl
from jax.experimental.pallas import tpu as pltpu
from jax.experimental.pallas import tpu_sc as plsc

NB_ATOMS = 512
SPLINE_SIZE = 61
SPLINE_RANGE = 2.0
BATCH = 16384
GRID = 2.0 * SPLINE_RANGE / (SPLINE_SIZE - 1)
HALF = SPLINE_SIZE // 2

NC = 2
NS = 16
NW = NC * NS
APW = NB_ATOMS // NW
TW = APW * SPLINE_SIZE
CW = 1024
NCHUNK = BATCH // CW
NPAIR = NCHUNK // 2
REPW = 16 * SPLINE_SIZE
TWREP = APW * REPW


def _forward(x, coefficients_vect):
    mesh = plsc.VectorSubcoreMesh(core_axis_name="c", subcore_axis_name="s")

    @functools.partial(
        pl.kernel,
        out_type=jax.ShapeDtypeStruct((NB_ATOMS, BATCH), jnp.float32),
        mesh=mesh,
        compiler_params=pltpu.CompilerParams(
            needs_layout_passes=False,
            disable_bounds_checks=True,
            skip_device_barrier=True),
        scratch_types=[
            pltpu.VMEM((TW,), jnp.float32),
            pltpu.VMEM((TW,), jnp.float32),
            pltpu.VMEM((TW,), jnp.float32),
            pltpu.VMEM((TWREP,), jnp.float32),
            pltpu.VMEM((TWREP,), jnp.float32),
            pltpu.VMEM((APW, CW), jnp.float32),
            pltpu.VMEM((APW, CW), jnp.float32),
            pltpu.VMEM((APW, CW), jnp.float32),
            pltpu.VMEM((APW, CW), jnp.float32),
            pltpu.SemaphoreType.DMA,
            pltpu.SemaphoreType.DMA,
            pltpu.SemaphoreType.DMA,
            pltpu.SemaphoreType.DMA,
        ],
    )
    def body(x_hbm, c_hbm, out_hbm, raw_v, proj_v, slp_v, arep_v, srep_v,
             xb0, xb1, ob0, ob1, si0, si1, so0, so1):
        wid = lax.axis_index("s") * NC + lax.axis_index("c")
        lanes = lax.iota(jnp.int32, 16)
        bi = lanes * SPLINE_SIZE
        rows = pl.ds(wid * APW, APW)

        pltpu.make_async_copy(
            x_hbm.at[rows, pl.ds(0, CW)], xb0, si0).start()

        pltpu.sync_copy(c_hbm.at[pl.ds(wid * TW, TW)], raw_v)

        zero = jnp.zeros((16,), jnp.float32)
        col0 = plsc.load_gather(raw_v, [bi])
        plsc.store_scatter(proj_v, [bi], zero)

        def pbody(j, c):
            col_prev, acc, sum_c, sum_p = c
            col = plsc.load_gather(raw_v, [bi + j])
            slope = jnp.minimum(jnp.maximum(col - col_prev, 0.0),
                                jnp.float32(GRID))
            acc = acc + slope
            plsc.store_scatter(proj_v, [bi + j], acc)
            plsc.store_scatter(slp_v, [bi + (j - 1)], slope)
            return (col, acc, sum_c + col, sum_p + acc)

        _, _, sum_c, sum_p = lax.fori_loop(
            1, SPLINE_SIZE, pbody, (col0, zero, col0, zero))
        mean = (sum_c - sum_p) * jnp.float32(1.0 / SPLINE_SIZE)

        bi16 = lanes * REPW
        perms = [bi16 + ((lanes + t) & 15) for t in range(16)]

        def abody(j, carry):
            v = plsc.load_gather(proj_v, [bi + j])
            s = plsc.load_gather(slp_v, [bi + j])
            jf = (j - HALF).astype(jnp.float32)
            a = v + mean - jf * s
            col = 16 * j
            for t in range(16):
                idx = col + perms[t]
                plsc.store_scatter(arep_v, [idx], a)
                plsc.store_scatter(srep_v, [idx], s)
            return carry

        lax.fori_loop(0, SPLINE_SIZE - 1, abody, 0)

        inv_g = jnp.float32(1.0 / GRID)
        q_lo = jnp.float32(np.float32(-(GRID * HALF)) / np.float32(GRID))
        q_hi = jnp.float32(28.75)

        def in_copy(ch, buf, sem):
            return pltpu.make_async_copy(
                x_hbm.at[rows, pl.ds(ch * CW, CW)], buf, sem)

        def out_copy(ch, buf, sem):
            return pltpu.make_async_copy(
                buf, out_hbm.at[rows, pl.ds(ch * CW, CW)], sem)

        def compute(xb, ob):
            def row_body(r, rcarry):
                lane_base = lanes + (
                    r * REPW + 16 * (HALF - 128) + 1342177280)

                @plsc.parallel_loop(0, CW, 16, unroll=8)
                def col_body(c0):
                    xv = xb[r, pl.ds(c0, 16)]
                    q = xv * inv_g
                    qc = jnp.minimum(jnp.maximum(q, q_lo), q_hi)
                    y = (qc + 127.5) + jnp.float32(8388608.0)
                    idx = (plsc.bitcast(y, jnp.int32) << 4) + lane_base
                    av = plsc.load_gather(arep_v, [idx])
                    sv = plsc.load_gather(srep_v, [idx])
                    ob[r, pl.ds(c0, 16)] = av + q * sv

                return rcarry

            lax.fori_loop(0, APW, row_body, 0)

        def pair_body(i, carry):
            c0 = 2 * i
            c1 = c0 + 1
            in_copy(c1, xb1, si1).start()
            in_copy(c0, xb0, si0).wait()

            @pl.when(i > 0)
            def _():
                out_copy(c0, ob0, so0).wait()

            compute(xb0, ob0)
            out_copy(c0, ob0, so0).start()

            @pl.when(i < NPAIR - 1)
            def _():
                in_copy(c0 + 2, xb0, si0).start()

            in_copy(c1, xb1, si1).wait()

            @pl.when(i > 0)
            def _():
                out_copy(c1, ob1, so1).wait()

            compute(xb1, ob1)
            out_copy(c1, ob1, so1).start()
            return carry

        lax.fori_loop(0, NPAIR, pair_body, 0)
        out_copy(NCHUNK - 2, ob0, so0).wait()
        out_copy(NCHUNK - 1, ob1, so1).wait()

    return body(x, coefficients_vect)


def kernel(x, coefficients_vect, L):
    del L
    return _forward(x, coefficients_vect)


# speedup vs baseline: 1.0611x; 1.0611x over previous
import functools

import jax
import jax.numpy as jnp
import numpy as np
from jax import lax
from jax.experimental import pallas as pl
from jax.experimental.pallas import tpu as pltpu
from jax.experimental.pallas import tpu_sc as plsc

NB_ATOMS = 512
SPLINE_SIZE = 61
SPLINE_RANGE = 2.0
BATCH = 16384
GRID = 2.0 * SPLINE_RANGE / (SPLINE_SIZE - 1)
HALF = SPLINE_SIZE // 2

NC = 2
NS = 16
NW = NC * NS
APW = NB_ATOMS // NW
TW = APW * SPLINE_SIZE
NCHUNK = APW
NPAIR = NCHUNK // 2
REPW = 16 * SPLINE_SIZE
TWREP = APW * REPW


def _forward(x, coefficients_vect):
    mesh = plsc.VectorSubcoreMesh(core_axis_name="c", subcore_axis_name="s")

    @functools.partial(
        pl.kernel,
        out_type=jax.ShapeDtypeStruct((NB_ATOMS, BATCH), jnp.float32),
        mesh=mesh,
        compiler_params=pltpu.CompilerParams(needs_layout_passes=False),
        scratch_types=[
            pltpu.VMEM((TW,), jnp.float32),
            pltpu.VMEM((TW,), jnp.float32),
            pltpu.VMEM((TW,), jnp.float32),
            pltpu.VMEM((TWREP,), jnp.float32),
            pltpu.VMEM((TWREP,), jnp.float32),
            pltpu.VMEM((BATCH,), jnp.float32),
            pltpu.VMEM((BATCH,), jnp.float32),
            pltpu.VMEM((BATCH,), jnp.float32),
            pltpu.VMEM((BATCH,), jnp.float32),
            pltpu.SemaphoreType.DMA,
            pltpu.SemaphoreType.DMA,
            pltpu.SemaphoreType.DMA,
            pltpu.SemaphoreType.DMA,
        ],
    )
    def body(x_hbm, c_hbm, out_hbm, raw_v, proj_v, slp_v, arep_v, srep_v,
             xb0, xb1, ob0, ob1, si0, si1, so0, so1):
        wid = lax.axis_index("s") * NC + lax.axis_index("c")
        lanes = lax.iota(jnp.int32, 16)
        bi = lanes * SPLINE_SIZE
        row0 = wid * APW

        pltpu.make_async_copy(x_hbm.at[row0], xb0, si0).start()

        pltpu.sync_copy(c_hbm.at[pl.ds(wid * TW, TW)], raw_v)

        zero = jnp.zeros((16,), jnp.float32)
        col0 = plsc.load_gather(raw_v, [bi])
        plsc.store_scatter(proj_v, [bi], zero)

        def pbody(j, c):
            col_prev, acc, sum_c, sum_p = c
            col = plsc.load_gather(raw_v, [bi + j])
            slope = jnp.minimum(jnp.maximum(col - col_prev, 0.0),
                                jnp.float32(GRID))
            acc = acc + slope
            plsc.store_scatter(proj_v, [bi + j], acc)
            plsc.store_scatter(slp_v, [bi + (j - 1)], slope)
            return (col, acc, sum_c + col, sum_p + acc)

        _, _, sum_c, sum_p = lax.fori_loop(
            1, SPLINE_SIZE, pbody, (col0, zero, col0, zero))
        mean = (sum_c - sum_p) * jnp.float32(1.0 / SPLINE_SIZE)

        bi16 = lanes * REPW
        perms = [bi16 + ((lanes + t) & 15) for t in range(16)]

        def abody(j, carry):
            v = plsc.load_gather(proj_v, [bi + j])
            s = plsc.load_gather(slp_v, [bi + j])
            jf = (j - HALF).astype(jnp.float32)
            a = v + mean - jf * s
            col = 16 * j
            for t in range(16):
                idx = col + perms[t]
                plsc.store_scatter(arep_v, [idx], a)
                plsc.store_scatter(srep_v, [idx], s)
            return carry

        lax.fori_loop(0, SPLINE_SIZE - 1, abody, 0)

        inv_g = jnp.float32(1.0 / GRID)
        q_lo = jnp.float32(np.float32(-(GRID * HALF)) / np.float32(GRID))
        q_hi = jnp.float32(28.75)

        def in_copy(r, buf, sem):
            return pltpu.make_async_copy(x_hbm.at[row0 + r], buf, sem)

        def out_copy(r, buf, sem):
            return pltpu.make_async_copy(buf, out_hbm.at[row0 + r], sem)

        def compute(r, xb, ob):
            if True:
                lane_base = lanes + (
                    r * REPW + 16 * (HALF - 128) + 1342177280)

                @plsc.parallel_loop(0, BATCH, 16, unroll=8)
                def col_body(c0):
                    xv = xb[pl.ds(c0, 16)]
                    q = xv * inv_g
                    qc = jnp.minimum(jnp.maximum(q, q_lo), q_hi)
                    y = (qc + 127.5) + jnp.float32(8388608.0)
                    idx = (plsc.bitcast(y, jnp.int32) << 4) + lane_base
                    av = plsc.load_gather(arep_v, [idx])
                    sv = plsc.load_gather(srep_v, [idx])
                    ob[pl.ds(c0, 16)] = av + q * sv

        def pair_body(i, carry):
            c0 = 2 * i
            c1 = c0 + 1
            in_copy(c1, xb1, si1).start()
            in_copy(c0, xb0, si0).wait()

            @pl.when(i > 0)
            def _():
                out_copy(c0, ob0, so0).wait()

            compute(c0, xb0, ob0)
            out_copy(c0, ob0, so0).start()

            @pl.when(i < NPAIR - 1)
            def _():
                in_copy(c0 + 2, xb0, si0).start()

            in_copy(c1, xb1, si1).wait()

            @pl.when(i > 0)
            def _():
                out_copy(c1, ob1, so1).wait()

            compute(c1, xb1, ob1)
            out_copy(c1, ob1, so1).start()
            return carry

        lax.fori_loop(0, NPAIR, pair_body, 0)
        out_copy(NCHUNK - 2, ob0, so0).wait()
        out_copy(NCHUNK - 1, ob1, so1).wait()

    return body(x, coefficients_vect)


def kernel(x, coefficients_vect, L):
    del L
    return _forward(x, coefficients_vect)


# speedup vs baseline: 1.0633x; 1.0020x over previous
import functools

import jax
import jax.numpy as jnp
import numpy as np
from jax import lax
from jax.experimental import pallas as pl
from jax.experimental.pallas import tpu as pltpu
from jax.experimental.pallas import tpu_sc as plsc

NB_ATOMS = 512
SPLINE_SIZE = 61
SPLINE_RANGE = 2.0
BATCH = 16384
GRID = 2.0 * SPLINE_RANGE / (SPLINE_SIZE - 1)
HALF = SPLINE_SIZE // 2

NC = 2
NS = 16
NW = NC * NS
APW = NB_ATOMS // NW
TW = APW * SPLINE_SIZE
NCHUNK = APW
NPAIR = NCHUNK // 2
REPW = 16 * SPLINE_SIZE
TWREP = APW * REPW


def _forward(x, coefficients_vect):
    mesh = plsc.VectorSubcoreMesh(core_axis_name="c", subcore_axis_name="s")

    @functools.partial(
        pl.kernel,
        out_type=jax.ShapeDtypeStruct((NB_ATOMS, BATCH), jnp.float32),
        mesh=mesh,
        compiler_params=pltpu.CompilerParams(needs_layout_passes=False),
        scratch_types=[
            pltpu.VMEM((TW,), jnp.float32),
            pltpu.VMEM((TW,), jnp.float32),
            pltpu.VMEM((TW,), jnp.float32),
            pltpu.VMEM((TWREP,), jnp.float32),
            pltpu.VMEM((TWREP,), jnp.float32),
            pltpu.VMEM((BATCH,), jnp.float32),
            pltpu.VMEM((BATCH,), jnp.float32),
            pltpu.VMEM((BATCH,), jnp.float32),
            pltpu.VMEM((BATCH,), jnp.float32),
            pltpu.SemaphoreType.DMA,
            pltpu.SemaphoreType.DMA,
            pltpu.SemaphoreType.DMA,
            pltpu.SemaphoreType.DMA,
        ],
    )
    def body(x_hbm, c_hbm, out_hbm, raw_v, proj_v, slp_v, arep_v, srep_v,
             xb0, xb1, ob0, ob1, si0, si1, so0, so1):
        wid = lax.axis_index("s") * NC + lax.axis_index("c")
        lanes = lax.iota(jnp.int32, 16)
        bi = lanes * SPLINE_SIZE
        row0 = wid * APW

        pltpu.make_async_copy(x_hbm.at[row0], xb0, si0).start()

        pltpu.sync_copy(c_hbm.at[pl.ds(wid * TW, TW)], raw_v)

        zero = jnp.zeros((16,), jnp.float32)
        col0 = plsc.load_gather(raw_v, [bi])
        plsc.store_scatter(proj_v, [bi], zero)

        def pbody(j, c):
            col_prev, acc, sum_c, sum_p = c
            col = plsc.load_gather(raw_v, [bi + j])
            slope = jnp.minimum(jnp.maximum(col - col_prev, 0.0),
                                jnp.float32(GRID))
            acc = acc + slope
            plsc.store_scatter(proj_v, [bi + j], acc)
            plsc.store_scatter(slp_v, [bi + (j - 1)], slope)
            return (col, acc, sum_c + col, sum_p + acc)

        _, _, sum_c, sum_p = lax.fori_loop(
            1, SPLINE_SIZE, pbody, (col0, zero, col0, zero))
        mean = (sum_c - sum_p) * jnp.float32(1.0 / SPLINE_SIZE)

        bi16 = lanes * REPW
        perms = [bi16 + ((lanes + t) & 15) for t in range(16)]

        def abody(j, carry):
            v = plsc.load_gather(proj_v, [bi + j])
            s = plsc.load_gather(slp_v, [bi + j])
            jf = (j - HALF).astype(jnp.float32)
            a = v + mean - jf * s
            col = 16 * j
            for t in range(16):
                idx = col + perms[t]
                plsc.store_scatter(arep_v, [idx], a)
                plsc.store_scatter(srep_v, [idx], s)
            return carry

        lax.fori_loop(0, SPLINE_SIZE - 1, abody, 0)

        inv_g = jnp.float32(1.0 / GRID)
        q_lo = jnp.float32(np.float32(-(GRID * HALF)) / np.float32(GRID))
        q_hi = jnp.float32(28.75)

        def in_copy(r, buf, sem):
            return pltpu.make_async_copy(x_hbm.at[row0 + r], buf, sem)

        def out_copy(r, buf, sem):
            return pltpu.make_async_copy(buf, out_hbm.at[row0 + r], sem)

        def compute(r, xb, ob):
            if True:
                lane_base = lanes + (
                    r * REPW + 16 * (HALF - 128) + 1342177280)

                @plsc.parallel_loop(0, BATCH, 16, unroll=4)
                def col_body(c0):
                    xv = xb[pl.ds(c0, 16)]
                    q = xv * inv_g
                    qc = jnp.minimum(jnp.maximum(q, q_lo), q_hi)
                    y = (qc + 127.5) + jnp.float32(8388608.0)
                    idx = (plsc.bitcast(y, jnp.int32) << 4) + lane_base
                    av = plsc.load_gather(arep_v, [idx])
                    sv = plsc.load_gather(srep_v, [idx])
                    ob[pl.ds(c0, 16)] = av + q * sv

        def pair_body(i, carry):
            c0 = 2 * i
            c1 = c0 + 1
            in_copy(c1, xb1, si1).start()
            in_copy(c0, xb0, si0).wait()

            @pl.when(i > 0)
            def _():
                out_copy(c0, ob0, so0).wait()

            compute(c0, xb0, ob0)
            out_copy(c0, ob0, so0).start()

            @pl.when(i < NPAIR - 1)
            def _():
                in_copy(c0 + 2, xb0, si0).start()

            in_copy(c1, xb1, si1).wait()

            @pl.when(i > 0)
            def _():
                out_copy(c1, ob1, so1).wait()

            compute(c1, xb1, ob1)
            out_copy(c1, ob1, so1).start()
            return carry

        lax.fori_loop(0, NPAIR, pair_body, 0)
        out_copy(NCHUNK - 2, ob0, so0).wait()
        out_copy(NCHUNK - 1, ob1, so1).wait()

    return body(x, coefficients_vect)


def kernel(x, coefficients_vect, L):
    del L
    return _forward(x, coefficients_vect)
